# baseline (speedup 1.0000x reference)
import functools
from math import factorial

import numpy as np
import jax
import jax.numpy as jnp
from jax import lax
from jax.experimental import pallas as pl
from jax.experimental.pallas import tpu as pltpu
from jax.experimental.pallas import tpu_sc as plsc

L_MAX = 3
LAMBDA_MAX = 3
N_ENV = 128
N_FEAT = 16


def _cg_table(l1, l2, lam):
    cg = np.zeros((2 * l1 + 1, 2 * l2 + 1, 2 * lam + 1), dtype=np.float64)
    for m1 in range(-l1, l1 + 1):
        for m2 in range(-l2, l2 + 1):
            mu = m1 + m2
            if abs(mu) > lam:
                continue
            pref = ((2 * lam + 1) * factorial(lam + l1 - l2) * factorial(lam - l1 + l2)
                    * factorial(l1 + l2 - lam) / factorial(l1 + l2 + lam + 1)) ** 0.5
            pref *= (factorial(lam + mu) * factorial(lam - mu) * factorial(l1 - m1)
                     * factorial(l1 + m1) * factorial(l2 - m2) * factorial(l2 + m2)) ** 0.5
            s = 0.0
            for k in range(0, l1 + l2 - lam + 1):
                dens = [k, l1 + l2 - lam - k, l1 - m1 - k, l2 + m2 - k,
                        lam - l2 + m1 + k, lam - l1 - m2 + k]
                if any(d < 0 for d in dens):
                    continue
                denom = 1.0
                for d in dens:
                    denom *= factorial(d)
                s += ((-1.0) ** k) / denom
            cg[m1 + l1, m2 + l2, mu + lam] = pref * s
    return cg

_MOFF = [0, 1, 4, 9]

def _lams_of(l1, l2):
    return list(range(abs(l1 - l2), min(l1 + l2, LAMBDA_MAX) + 1))

_PAIR_LIST = [(l1, l2) for l1 in range(L_MAX + 1) for l2 in range(L_MAX + 1)]
_PAIRS_LAM = {lam: [(l1, l2) for (l1, l2) in _PAIR_LIST if lam in _lams_of(l1, l2)]
              for lam in range(LAMBDA_MAX + 1)}
_N_PAIRS = [len(_PAIRS_LAM[lam]) for lam in range(4)]
_F_SIZES = [2 * _N_PAIRS[lam] * 256 for lam in range(4)]
_FT = [f // 128 for f in _F_SIZES]

_EVENTS = {}
for (_l1, _l2) in _PAIR_LIST:
    _per = {}
    for _lam in _lams_of(_l1, _l2):
        _cg = _cg_table(_l1, _l2, _lam)
        _evs = []
        for _k in range(2 * _lam + 1):
            for _m in range(2 * _l1 + 1):
                _n = (_k - _lam) - (_m - _l1) + _l2
                if 0 <= _n <= 2 * _l2 and _cg[_m, _n, _k] != 0.0:
                    _evs.append((_k, _m, _n, float(_cg[_m, _n, _k])))
        _per[_lam] = _evs
    _EVENTS[(_l1, _l2)] = _per

_STEPS = [(ch, pr) for ch in range(2) for pr in range(len(_PAIR_LIST))]
_CHUNK_WORDS = 2048

def _pair_sections(l1, l2):
    secs = []
    off = 0
    for lam in _lams_of(l1, l2):
        if lam == 0:
            continue
        for k in range(2 * lam + 1):
            secs.append((lam, k, off))
            off += _CHUNK_WORDS
    return secs

_MAX_CHUNK = max(
    len(_pair_sections(l1, l2)) for (l1, l2) in _PAIR_LIST) * _CHUNK_WORDS


def _tec_kernel(in_ref, oe0, oe1, oe2, oe3, oo0, oo1, oo2, oo3,
                in_v, chunk0, chunk1, lam0a, lam0b, dma_sem):
    out_even = [oe0, oe1, oe2, oe3]
    out_odd = [oo0, oo1, oo2, oo3]
    chunks = [chunk0, chunk1]
    lam0_bufs = [lam0a, lam0b]

    cid = lax.axis_index("c")
    sid = lax.axis_index("s")
    wid = sid * 2 + cid
    e_tile = wid // 2
    c_half = wid % 2
    env_base = e_tile * 8

    pltpu.sync_copy(in_ref.at[pl.ds(env_base * 1024, 8 * 1024)], in_v)

    et_off = [e_tile * _FT[lam] * 1024 for lam in range(4)]
    ch_off = [c_half * _N_PAIRS[lam] * 2048 for lam in range(4)]
    a_off = c_half * 256

    def make_pair_body(l1, l2, parity, buf, lam0_v):
        evs_by_lamk = {}
        for lam in _lams_of(l1, l2):
            for (ke, m, n, c) in _EVENTS[(l1, l2)][lam]:
                evs_by_lamk.setdefault((lam, ke), []).append((m, n, c))
        sec_off = {(lam, k): off for (lam, k, off) in _pair_sections(l1, l2)}
        p_lam0 = None
        if 0 in _lams_of(l1, l2):
            p_lam0 = _PAIRS_LAM[0].index((l1, l2))
        if parity == 0:
            b_par = a_off
        else:
            b_par = 256 - a_off

        def f_body(f, e_local):
            in_base = e_local * 1024
            a_row = in_v[pl.ds(in_base + a_off + f * 16, 16)]
            a_sp = {}
            for m in range(2 * l1 + 1):
                a_sp[m] = jnp.full((16,), a_row[_MOFF[l1] + m], jnp.float32)
            b_base = in_base + 512 + b_par
            b_rows = [in_v[pl.ds(b_base + (_MOFF[l2] + n) * 16, 16)]
                      for n in range(2 * l2 + 1)]
            prod = {}
            ft = f // 8
            fo = (f % 8) * 16
            for lam in _lams_of(l1, l2):
                for k in range(2 * lam + 1):
                    acc = None
                    for (m, n, c) in evs_by_lamk[(lam, k)]:
                        t = prod.get((m, n))
                        if t is None:
                            t = a_sp[m] * b_rows[n]
                            prod[(m, n)] = t
                        term = t * np.float32(c)
                        acc = term if acc is None else acc + term
                    if lam == 0:
                        lam0_v[e_local, pl.ds(p_lam0 * 256 + f * 16, 16)] = acc
                    else:
                        off = sec_off[(lam, k)] + (ft * 8 + e_local) * 128 + fo
                        buf[pl.ds(off, 16)] = acc
            return e_local

        size = (2 * l1 + 1) * (2 * l2 + 1)
        unroll = 4 if size <= 9 else 2

        def e_body(e_local, carry):
            lax.fori_loop(0, 16, f_body, e_local, unroll=unroll)
            return carry

        lax.fori_loop(0, 8, e_body, 0)

    def issue_pair_dmas(l1, l2, parity, buf):
        outs = out_even if parity == 0 else out_odd
        for (lam, k, off) in _pair_sections(l1, l2):
            pos = _PAIRS_LAM[lam].index((l1, l2))
            dst = (k * 16 * _FT[lam] + 2 * pos) * 1024 + et_off[lam] + ch_off[lam]
            pltpu.async_copy(
                buf.at[pl.ds(off, _CHUNK_WORDS)],
                outs[lam].at[pl.ds(dst, _CHUNK_WORDS)],
                dma_sem)

    def drain_pair(l1, l2):
        n = len(_pair_sections(l1, l2))
        for _ in range(n):
            pltpu.make_async_copy(
                out_even[3].at[pl.ds(0, _CHUNK_WORDS)],
                chunk0.at[pl.ds(0, _CHUNK_WORDS)], dma_sem).wait()

    for i, (parity, pr) in enumerate(_STEPS):
        l1, l2 = _PAIR_LIST[pr]
        if i >= 2:
            pl1, pl2 = _PAIR_LIST[_STEPS[i - 2][1]]
            drain_pair(pl1, pl2)
        buf = chunks[i % 2]
        make_pair_body(l1, l2, parity, buf, lam0_bufs[parity])
        issue_pair_dmas(l1, l2, parity, buf)
    for i in (len(_STEPS) - 2, len(_STEPS) - 1):
        pl1, pl2 = _PAIR_LIST[_STEPS[i][1]]
        drain_pair(pl1, pl2)

    for e_local in range(8):
        dst = (env_base + e_local) * 2048 + c_half * 1024
        pltpu.async_copy(lam0a.at[e_local], oe0.at[pl.ds(dst, 1024)], dma_sem)
        pltpu.async_copy(lam0b.at[e_local], oo0.at[pl.ds(dst, 1024)], dma_sem)
    for _ in range(16):
        pltpu.make_async_copy(
            oe0.at[pl.ds(0, 1024)], lam0a.at[0], dma_sem).wait()


@jax.jit
def _run_sc(packed):
    mesh = plsc.VectorSubcoreMesh(core_axis_name="c", subcore_axis_name="s")
    out_type = tuple(
        jax.ShapeDtypeStruct(((2 * lam + 1) * N_ENV * _F_SIZES[lam],), jnp.float32)
        for lam in range(4)) * 2
    scratch = [
        pltpu.VMEM((8 * 1024,), jnp.float32),
        pltpu.VMEM((_MAX_CHUNK,), jnp.float32),
        pltpu.VMEM((_MAX_CHUNK,), jnp.float32),
        pltpu.VMEM((8, 1024), jnp.float32),
        pltpu.VMEM((8, 1024), jnp.float32),
        pltpu.SemaphoreType.DMA,
    ]
    fn = pl.kernel(_tec_kernel, out_type=out_type, mesh=mesh,
                   scratch_types=scratch,
                   compiler_params=pltpu.CompilerParams(
                       needs_layout_passes=False))
    return fn(packed.reshape(-1))


def kernel(first_even_0, first_even_1, first_even_2, first_even_3,
           first_odd_0, first_odd_1, first_odd_2, first_odd_3,
           second_even_0, second_even_1, second_even_2, second_even_3,
           second_odd_0, second_odd_1, second_odd_2, second_odd_3):
    first_e = jnp.concatenate(
        [first_even_0, first_even_1, first_even_2, first_even_3], axis=2)
    first_o = jnp.concatenate(
        [first_odd_0, first_odd_1, first_odd_2, first_odd_3], axis=2)
    sec_e = jnp.concatenate(
        [second_even_0, second_even_1, second_even_2, second_even_3],
        axis=2).transpose(0, 2, 1)
    sec_o = jnp.concatenate(
        [second_odd_0, second_odd_1, second_odd_2, second_odd_3],
        axis=2).transpose(0, 2, 1)
    packed = jnp.concatenate(
        [first_e.reshape(N_ENV, 256), first_o.reshape(N_ENV, 256),
         sec_e.reshape(N_ENV, 256), sec_o.reshape(N_ENV, 256)], axis=1)

    outs = _run_sc(packed)
    res = [None] * 8
    for half in range(2):
        res[half * 4] = outs[half * 4].reshape(N_ENV, _F_SIZES[0], 1)
        for lam in range(1, 4):
            F = _F_SIZES[lam]
            K = 2 * lam + 1
            x5 = outs[half * 4 + lam].reshape(K, 16, F // 128, 8, 128)
            res[half * 4 + lam] = x5.transpose(1, 3, 2, 4, 0).reshape(
                N_ENV, F, K)
    return tuple(res)


# speedup vs baseline: 1.0087x; 1.0087x over previous
import functools
from math import factorial

import numpy as np
import jax
import jax.numpy as jnp
from jax import lax
from jax.experimental import pallas as pl
from jax.experimental.pallas import tpu as pltpu
from jax.experimental.pallas import tpu_sc as plsc

L_MAX = 3
LAMBDA_MAX = 3
N_ENV = 128
N_FEAT = 16


def _cg_table(l1, l2, lam):
    cg = np.zeros((2 * l1 + 1, 2 * l2 + 1, 2 * lam + 1), dtype=np.float64)
    for m1 in range(-l1, l1 + 1):
        for m2 in range(-l2, l2 + 1):
            mu = m1 + m2
            if abs(mu) > lam:
                continue
            pref = ((2 * lam + 1) * factorial(lam + l1 - l2) * factorial(lam - l1 + l2)
                    * factorial(l1 + l2 - lam) / factorial(l1 + l2 + lam + 1)) ** 0.5
            pref *= (factorial(lam + mu) * factorial(lam - mu) * factorial(l1 - m1)
                     * factorial(l1 + m1) * factorial(l2 - m2) * factorial(l2 + m2)) ** 0.5
            s = 0.0
            for k in range(0, l1 + l2 - lam + 1):
                dens = [k, l1 + l2 - lam - k, l1 - m1 - k, l2 + m2 - k,
                        lam - l2 + m1 + k, lam - l1 - m2 + k]
                if any(d < 0 for d in dens):
                    continue
                denom = 1.0
                for d in dens:
                    denom *= factorial(d)
                s += ((-1.0) ** k) / denom
            cg[m1 + l1, m2 + l2, mu + lam] = pref * s
    return cg

_MOFF = [0, 1, 4, 9]

def _lams_of(l1, l2):
    return list(range(abs(l1 - l2), min(l1 + l2, LAMBDA_MAX) + 1))

_PAIR_LIST = [(l1, l2) for l1 in range(L_MAX + 1) for l2 in range(L_MAX + 1)]
_PAIRS_LAM = {lam: [(l1, l2) for (l1, l2) in _PAIR_LIST if lam in _lams_of(l1, l2)]
              for lam in range(LAMBDA_MAX + 1)}
_N_PAIRS = [len(_PAIRS_LAM[lam]) for lam in range(4)]
_F_SIZES = [2 * _N_PAIRS[lam] * 256 for lam in range(4)]
_FT = [f // 128 for f in _F_SIZES]

_EVENTS = {}
for (_l1, _l2) in _PAIR_LIST:
    _per = {}
    for _lam in _lams_of(_l1, _l2):
        _cg = _cg_table(_l1, _l2, _lam)
        _evs = []
        for _k in range(2 * _lam + 1):
            for _m in range(2 * _l1 + 1):
                _n = (_k - _lam) - (_m - _l1) + _l2
                if 0 <= _n <= 2 * _l2 and _cg[_m, _n, _k] != 0.0:
                    _evs.append((_k, _m, _n, float(_cg[_m, _n, _k])))
        _per[_lam] = _evs
    _EVENTS[(_l1, _l2)] = _per

_STEPS = [(ch, pr) for ch in range(2) for pr in range(len(_PAIR_LIST))]
_CHUNK_WORDS = 2048

def _pair_sections(l1, l2):
    secs = []
    off = 0
    for lam in _lams_of(l1, l2):
        if lam == 0:
            continue
        for k in range(2 * lam + 1):
            secs.append((lam, k, off))
            off += _CHUNK_WORDS
    return secs

_MAX_CHUNK = max(
    len(_pair_sections(l1, l2)) for (l1, l2) in _PAIR_LIST) * _CHUNK_WORDS


def _tec_kernel(in_ref, oe0, oe1, oe2, oe3, oo0, oo1, oo2, oo3,
                in_v, chunk0, chunk1, lam0a, lam0b, dma_sem):
    out_even = [oe0, oe1, oe2, oe3]
    out_odd = [oo0, oo1, oo2, oo3]
    chunks = [chunk0, chunk1]
    lam0_bufs = [lam0a, lam0b]

    cid = lax.axis_index("c")
    sid = lax.axis_index("s")
    wid = sid * 2 + cid
    e_tile = wid // 2
    c_half = wid % 2
    env_base = e_tile * 8

    pltpu.sync_copy(in_ref.at[pl.ds(env_base * 1024, 8 * 1024)], in_v)

    et_off = [e_tile * _FT[lam] * 1024 for lam in range(4)]
    ch_off = [c_half * _N_PAIRS[lam] * 2048 for lam in range(4)]
    a_off = c_half * 256

    def make_pair_body(l1, l2, parity, buf, lam0_v):
        evs_by_lamk = {}
        for lam in _lams_of(l1, l2):
            for (ke, m, n, c) in _EVENTS[(l1, l2)][lam]:
                evs_by_lamk.setdefault((lam, ke), []).append((m, n, c))
        sec_off = {(lam, k): off for (lam, k, off) in _pair_sections(l1, l2)}
        p_lam0 = None
        if 0 in _lams_of(l1, l2):
            p_lam0 = _PAIRS_LAM[0].index((l1, l2))
        if parity == 0:
            b_par = a_off
        else:
            b_par = 256 - a_off

        def f_body(f, e_local):
            in_base = e_local * 1024
            a_row = in_v[pl.ds(in_base + a_off + f * 16, 16)]
            a_sp = {}
            for m in range(2 * l1 + 1):
                a_sp[m] = jnp.full((16,), a_row[_MOFF[l1] + m], jnp.float32)
            b_base = in_base + 512 + b_par
            b_rows = [in_v[pl.ds(b_base + (_MOFF[l2] + n) * 16, 16)]
                      for n in range(2 * l2 + 1)]
            prod = {}
            ft = f // 8
            fo = (f % 8) * 16
            for lam in _lams_of(l1, l2):
                for k in range(2 * lam + 1):
                    acc = None
                    for (m, n, c) in evs_by_lamk[(lam, k)]:
                        t = prod.get((m, n))
                        if t is None:
                            t = a_sp[m] * b_rows[n]
                            prod[(m, n)] = t
                        term = t * np.float32(c)
                        acc = term if acc is None else acc + term
                    if lam == 0:
                        lam0_v[e_local, pl.ds(p_lam0 * 256 + f * 16, 16)] = acc
                    else:
                        off = sec_off[(lam, k)] + (ft * 8 + e_local) * 128 + fo
                        buf[pl.ds(off, 16)] = acc
            return e_local

        size = (2 * l1 + 1) * (2 * l2 + 1)
        unroll = 2 if size <= 9 else 1

        def e_body(e_local, carry):
            lax.fori_loop(0, 16, f_body, e_local, unroll=unroll)
            return carry

        lax.fori_loop(0, 8, e_body, 0)

    def issue_pair_dmas(l1, l2, parity, buf):
        outs = out_even if parity == 0 else out_odd
        for (lam, k, off) in _pair_sections(l1, l2):
            pos = _PAIRS_LAM[lam].index((l1, l2))
            dst = (k * 16 * _FT[lam] + 2 * pos) * 1024 + et_off[lam] + ch_off[lam]
            pltpu.async_copy(
                buf.at[pl.ds(off, _CHUNK_WORDS)],
                outs[lam].at[pl.ds(dst, _CHUNK_WORDS)],
                dma_sem)

    def drain_pair(l1, l2):
        n = len(_pair_sections(l1, l2))
        for _ in range(n):
            pltpu.make_async_copy(
                out_even[3].at[pl.ds(0, _CHUNK_WORDS)],
                chunk0.at[pl.ds(0, _CHUNK_WORDS)], dma_sem).wait()

    for i, (parity, pr) in enumerate(_STEPS):
        l1, l2 = _PAIR_LIST[pr]
        if i >= 2:
            pl1, pl2 = _PAIR_LIST[_STEPS[i - 2][1]]
            drain_pair(pl1, pl2)
        buf = chunks[i % 2]
        make_pair_body(l1, l2, parity, buf, lam0_bufs[parity])
        issue_pair_dmas(l1, l2, parity, buf)
    for i in (len(_STEPS) - 2, len(_STEPS) - 1):
        pl1, pl2 = _PAIR_LIST[_STEPS[i][1]]
        drain_pair(pl1, pl2)

    for e_local in range(8):
        dst = (env_base + e_local) * 2048 + c_half * 1024
        pltpu.async_copy(lam0a.at[e_local], oe0.at[pl.ds(dst, 1024)], dma_sem)
        pltpu.async_copy(lam0b.at[e_local], oo0.at[pl.ds(dst, 1024)], dma_sem)
    for _ in range(16):
        pltpu.make_async_copy(
            oe0.at[pl.ds(0, 1024)], lam0a.at[0], dma_sem).wait()


@jax.jit
def _run_sc(packed):
    mesh = plsc.VectorSubcoreMesh(core_axis_name="c", subcore_axis_name="s")
    out_type = tuple(
        jax.ShapeDtypeStruct(((2 * lam + 1) * N_ENV * _F_SIZES[lam],), jnp.float32)
        for lam in range(4)) * 2
    scratch = [
        pltpu.VMEM((8 * 1024,), jnp.float32),
        pltpu.VMEM((_MAX_CHUNK,), jnp.float32),
        pltpu.VMEM((_MAX_CHUNK,), jnp.float32),
        pltpu.VMEM((8, 1024), jnp.float32),
        pltpu.VMEM((8, 1024), jnp.float32),
        pltpu.SemaphoreType.DMA,
    ]
    fn = pl.kernel(_tec_kernel, out_type=out_type, mesh=mesh,
                   scratch_types=scratch,
                   compiler_params=pltpu.CompilerParams(
                       needs_layout_passes=False))
    return fn(packed.reshape(-1))


def kernel(first_even_0, first_even_1, first_even_2, first_even_3,
           first_odd_0, first_odd_1, first_odd_2, first_odd_3,
           second_even_0, second_even_1, second_even_2, second_even_3,
           second_odd_0, second_odd_1, second_odd_2, second_odd_3):
    first_e = jnp.concatenate(
        [first_even_0, first_even_1, first_even_2, first_even_3], axis=2)
    first_o = jnp.concatenate(
        [first_odd_0, first_odd_1, first_odd_2, first_odd_3], axis=2)
    sec_e = jnp.concatenate(
        [second_even_0, second_even_1, second_even_2, second_even_3],
        axis=2).transpose(0, 2, 1)
    sec_o = jnp.concatenate(
        [second_odd_0, second_odd_1, second_odd_2, second_odd_3],
        axis=2).transpose(0, 2, 1)
    packed = jnp.concatenate(
        [first_e.reshape(N_ENV, 256), first_o.reshape(N_ENV, 256),
         sec_e.reshape(N_ENV, 256), sec_o.reshape(N_ENV, 256)], axis=1)

    outs = _run_sc(packed)
    res = [None] * 8
    for half in range(2):
        res[half * 4] = outs[half * 4].reshape(N_ENV, _F_SIZES[0], 1)
        for lam in range(1, 4):
            F = _F_SIZES[lam]
            K = 2 * lam + 1
            x5 = outs[half * 4 + lam].reshape(K, 16, F // 128, 8, 128)
            res[half * 4 + lam] = x5.transpose(1, 3, 2, 4, 0).reshape(
                N_ENV, F, K)
    return tuple(res)


# speedup vs baseline: 1.2561x; 1.2452x over previous
import functools
from math import factorial

import numpy as np
import jax
import jax.numpy as jnp
from jax import lax
from jax.experimental import pallas as pl
from jax.experimental.pallas import tpu as pltpu
from jax.experimental.pallas import tpu_sc as plsc

L_MAX = 3
LAMBDA_MAX = 3
N_ENV = 128
N_FEAT = 16


def _cg_table(l1, l2, lam):
    cg = np.zeros((2 * l1 + 1, 2 * l2 + 1, 2 * lam + 1), dtype=np.float64)
    for m1 in range(-l1, l1 + 1):
        for m2 in range(-l2, l2 + 1):
            mu = m1 + m2
            if abs(mu) > lam:
                continue
            pref = ((2 * lam + 1) * factorial(lam + l1 - l2) * factorial(lam - l1 + l2)
                    * factorial(l1 + l2 - lam) / factorial(l1 + l2 + lam + 1)) ** 0.5
            pref *= (factorial(lam + mu) * factorial(lam - mu) * factorial(l1 - m1)
                     * factorial(l1 + m1) * factorial(l2 - m2) * factorial(l2 + m2)) ** 0.5
            s = 0.0
            for k in range(0, l1 + l2 - lam + 1):
                dens = [k, l1 + l2 - lam - k, l1 - m1 - k, l2 + m2 - k,
                        lam - l2 + m1 + k, lam - l1 - m2 + k]
                if any(d < 0 for d in dens):
                    continue
                denom = 1.0
                for d in dens:
                    denom *= factorial(d)
                s += ((-1.0) ** k) / denom
            cg[m1 + l1, m2 + l2, mu + lam] = pref * s
    return cg

_MOFF = [0, 1, 4, 9]

def _lams_of(l1, l2):
    return list(range(abs(l1 - l2), min(l1 + l2, LAMBDA_MAX) + 1))

_PAIR_LIST = [(l1, l2) for l1 in range(L_MAX + 1) for l2 in range(L_MAX + 1)]
_PAIRS_LAM = {lam: [(l1, l2) for (l1, l2) in _PAIR_LIST if lam in _lams_of(l1, l2)]
              for lam in range(LAMBDA_MAX + 1)}
_N_PAIRS = [len(_PAIRS_LAM[lam]) for lam in range(4)]
_F_SIZES = [2 * _N_PAIRS[lam] * 256 for lam in range(4)]
_FT = [f // 128 for f in _F_SIZES]

_EVENTS = {}
for (_l1, _l2) in _PAIR_LIST:
    _per = {}
    for _lam in _lams_of(_l1, _l2):
        _cg = _cg_table(_l1, _l2, _lam)
        _evs = []
        for _k in range(2 * _lam + 1):
            for _m in range(2 * _l1 + 1):
                _n = (_k - _lam) - (_m - _l1) + _l2
                if 0 <= _n <= 2 * _l2 and _cg[_m, _n, _k] != 0.0:
                    _evs.append((_k, _m, _n, float(_cg[_m, _n, _k])))
        _per[_lam] = _evs
    _EVENTS[(_l1, _l2)] = _per

_STEPS = [(ch, pr) for ch in range(2) for pr in range(len(_PAIR_LIST))]
_CHUNK_WORDS = 2048

def _pair_sections(l1, l2):
    secs = []
    off = 0
    for lam in _lams_of(l1, l2):
        if lam == 0:
            continue
        for k in range(2 * lam + 1):
            secs.append((lam, k, off))
            off += _CHUNK_WORDS
    return secs

_MAX_CHUNK = max(
    len(_pair_sections(l1, l2)) for (l1, l2) in _PAIR_LIST) * _CHUNK_WORDS


def _tec_kernel(in_ref, oe0, oe1, oe2, oe3, oo0, oo1, oo2, oo3,
                in_v, chunk0, chunk1, lam0a, lam0b, dma_sem):
    out_even = [oe0, oe1, oe2, oe3]
    out_odd = [oo0, oo1, oo2, oo3]
    chunks = [chunk0, chunk1]
    lam0_bufs = [lam0a, lam0b]

    cid = lax.axis_index("c")
    sid = lax.axis_index("s")
    wid = sid * 2 + cid
    e_tile = wid // 2
    c_half = wid % 2
    env_base = e_tile * 8

    pltpu.sync_copy(in_ref.at[pl.ds(env_base * 1024, 8 * 1024)], in_v)

    et_off = [e_tile * _FT[lam] * 1024 for lam in range(4)]
    ch_off = [c_half * _N_PAIRS[lam] * 2048 for lam in range(4)]
    a_off = c_half * 256

    def make_pair_body(l1, l2, parity, buf, lam0_v):
        evs_by_lamk = {}
        for lam in _lams_of(l1, l2):
            for (ke, m, n, c) in _EVENTS[(l1, l2)][lam]:
                evs_by_lamk.setdefault((lam, ke), []).append((m, n, c))
        sec_off = {(lam, k): off for (lam, k, off) in _pair_sections(l1, l2)}
        p_lam0 = None
        if 0 in _lams_of(l1, l2):
            p_lam0 = _PAIRS_LAM[0].index((l1, l2))
        if parity == 0:
            b_par = a_off
        else:
            b_par = 256 - a_off

        def f_body(f, e_local):
            in_base = e_local * 1024
            a_row = in_v[pl.ds(in_base + a_off + f * 16, 16)]
            a_sp = {}
            for m in range(2 * l1 + 1):
                a_sp[m] = jnp.full((16,), a_row[_MOFF[l1] + m], jnp.float32)
            b_base = in_base + 512 + b_par
            b_rows = [in_v[pl.ds(b_base + (_MOFF[l2] + n) * 16, 16)]
                      for n in range(2 * l2 + 1)]
            prod = {}
            ft = f // 8
            fo = (f % 8) * 16
            for lam in _lams_of(l1, l2):
                for k in range(2 * lam + 1):
                    acc = None
                    for (m, n, c) in evs_by_lamk[(lam, k)]:
                        t = prod.get((m, n))
                        if t is None:
                            t = a_sp[m] * b_rows[n]
                            prod[(m, n)] = t
                        term = t * np.float32(c)
                        acc = term if acc is None else acc + term
                    if lam == 0:
                        lam0_v[e_local, pl.ds(p_lam0 * 256 + f * 16, 16)] = acc
                    else:
                        off = sec_off[(lam, k)] + (ft * 8 + e_local) * 128 + fo
                        buf[pl.ds(off, 16)] = acc
            return e_local

        def e_body(e_local, carry):
            plsc.parallel_loop(0, 16, carry=e_local)(f_body)
            return carry

        lax.fori_loop(0, 8, e_body, 0)

    def issue_pair_dmas(l1, l2, parity, buf):
        outs = out_even if parity == 0 else out_odd
        for (lam, k, off) in _pair_sections(l1, l2):
            pos = _PAIRS_LAM[lam].index((l1, l2))
            dst = (k * 16 * _FT[lam] + 2 * pos) * 1024 + et_off[lam] + ch_off[lam]
            pltpu.async_copy(
                buf.at[pl.ds(off, _CHUNK_WORDS)],
                outs[lam].at[pl.ds(dst, _CHUNK_WORDS)],
                dma_sem)

    def drain_pair(l1, l2):
        n = len(_pair_sections(l1, l2))
        for _ in range(n):
            pltpu.make_async_copy(
                out_even[3].at[pl.ds(0, _CHUNK_WORDS)],
                chunk0.at[pl.ds(0, _CHUNK_WORDS)], dma_sem).wait()

    for i, (parity, pr) in enumerate(_STEPS):
        l1, l2 = _PAIR_LIST[pr]
        if i >= 2:
            pl1, pl2 = _PAIR_LIST[_STEPS[i - 2][1]]
            drain_pair(pl1, pl2)
        buf = chunks[i % 2]
        make_pair_body(l1, l2, parity, buf, lam0_bufs[parity])
        issue_pair_dmas(l1, l2, parity, buf)
    for i in (len(_STEPS) - 2, len(_STEPS) - 1):
        pl1, pl2 = _PAIR_LIST[_STEPS[i][1]]
        drain_pair(pl1, pl2)

    for e_local in range(8):
        dst = (env_base + e_local) * 2048 + c_half * 1024
        pltpu.async_copy(lam0a.at[e_local], oe0.at[pl.ds(dst, 1024)], dma_sem)
        pltpu.async_copy(lam0b.at[e_local], oo0.at[pl.ds(dst, 1024)], dma_sem)
    for _ in range(16):
        pltpu.make_async_copy(
            oe0.at[pl.ds(0, 1024)], lam0a.at[0], dma_sem).wait()


@jax.jit
def _run_sc(packed):
    mesh = plsc.VectorSubcoreMesh(core_axis_name="c", subcore_axis_name="s")
    out_type = tuple(
        jax.ShapeDtypeStruct(((2 * lam + 1) * N_ENV * _F_SIZES[lam],), jnp.float32)
        for lam in range(4)) * 2
    scratch = [
        pltpu.VMEM((8 * 1024,), jnp.float32),
        pltpu.VMEM((_MAX_CHUNK,), jnp.float32),
        pltpu.VMEM((_MAX_CHUNK,), jnp.float32),
        pltpu.VMEM((8, 1024), jnp.float32),
        pltpu.VMEM((8, 1024), jnp.float32),
        pltpu.SemaphoreType.DMA,
    ]
    fn = pl.kernel(_tec_kernel, out_type=out_type, mesh=mesh,
                   scratch_types=scratch,
                   compiler_params=pltpu.CompilerParams(
                       needs_layout_passes=False))
    return fn(packed.reshape(-1))


def kernel(first_even_0, first_even_1, first_even_2, first_even_3,
           first_odd_0, first_odd_1, first_odd_2, first_odd_3,
           second_even_0, second_even_1, second_even_2, second_even_3,
           second_odd_0, second_odd_1, second_odd_2, second_odd_3):
    first_e = jnp.concatenate(
        [first_even_0, first_even_1, first_even_2, first_even_3], axis=2)
    first_o = jnp.concatenate(
        [first_odd_0, first_odd_1, first_odd_2, first_odd_3], axis=2)
    sec_e = jnp.concatenate(
        [second_even_0, second_even_1, second_even_2, second_even_3],
        axis=2).transpose(0, 2, 1)
    sec_o = jnp.concatenate(
        [second_odd_0, second_odd_1, second_odd_2, second_odd_3],
        axis=2).transpose(0, 2, 1)
    packed = jnp.concatenate(
        [first_e.reshape(N_ENV, 256), first_o.reshape(N_ENV, 256),
         sec_e.reshape(N_ENV, 256), sec_o.reshape(N_ENV, 256)], axis=1)

    outs = _run_sc(packed)
    res = [None] * 8
    for half in range(2):
        res[half * 4] = outs[half * 4].reshape(N_ENV, _F_SIZES[0], 1)
        for lam in range(1, 4):
            F = _F_SIZES[lam]
            K = 2 * lam + 1
            x5 = outs[half * 4 + lam].reshape(K, 16, F // 128, 8, 128)
            res[half * 4 + lam] = x5.transpose(1, 3, 2, 4, 0).reshape(
                N_ENV, F, K)
    return tuple(res)


# speedup vs baseline: 1.2574x; 1.0011x over previous
import functools
from math import factorial

import numpy as np
import jax
import jax.numpy as jnp
from jax import lax
from jax.experimental import pallas as pl
from jax.experimental.pallas import tpu as pltpu
from jax.experimental.pallas import tpu_sc as plsc

L_MAX = 3
LAMBDA_MAX = 3
N_ENV = 128
N_FEAT = 16


def _cg_table(l1, l2, lam):
    cg = np.zeros((2 * l1 + 1, 2 * l2 + 1, 2 * lam + 1), dtype=np.float64)
    for m1 in range(-l1, l1 + 1):
        for m2 in range(-l2, l2 + 1):
            mu = m1 + m2
            if abs(mu) > lam:
                continue
            pref = ((2 * lam + 1) * factorial(lam + l1 - l2) * factorial(lam - l1 + l2)
                    * factorial(l1 + l2 - lam) / factorial(l1 + l2 + lam + 1)) ** 0.5
            pref *= (factorial(lam + mu) * factorial(lam - mu) * factorial(l1 - m1)
                     * factorial(l1 + m1) * factorial(l2 - m2) * factorial(l2 + m2)) ** 0.5
            s = 0.0
            for k in range(0, l1 + l2 - lam + 1):
                dens = [k, l1 + l2 - lam - k, l1 - m1 - k, l2 + m2 - k,
                        lam - l2 + m1 + k, lam - l1 - m2 + k]
                if any(d < 0 for d in dens):
                    continue
                denom = 1.0
                for d in dens:
                    denom *= factorial(d)
                s += ((-1.0) ** k) / denom
            cg[m1 + l1, m2 + l2, mu + lam] = pref * s
    return cg

_MOFF = [0, 1, 4, 9]

def _lams_of(l1, l2):
    return list(range(abs(l1 - l2), min(l1 + l2, LAMBDA_MAX) + 1))

_PAIR_LIST = [(l1, l2) for l1 in range(L_MAX + 1) for l2 in range(L_MAX + 1)]
_PAIRS_LAM = {lam: [(l1, l2) for (l1, l2) in _PAIR_LIST if lam in _lams_of(l1, l2)]
              for lam in range(LAMBDA_MAX + 1)}
_N_PAIRS = [len(_PAIRS_LAM[lam]) for lam in range(4)]
_F_SIZES = [2 * _N_PAIRS[lam] * 256 for lam in range(4)]
_FT = [f // 128 for f in _F_SIZES]

_EVENTS = {}
for (_l1, _l2) in _PAIR_LIST:
    _per = {}
    for _lam in _lams_of(_l1, _l2):
        _cg = _cg_table(_l1, _l2, _lam)
        _evs = []
        for _k in range(2 * _lam + 1):
            for _m in range(2 * _l1 + 1):
                _n = (_k - _lam) - (_m - _l1) + _l2
                if 0 <= _n <= 2 * _l2 and _cg[_m, _n, _k] != 0.0:
                    _evs.append((_k, _m, _n, float(_cg[_m, _n, _k])))
        _per[_lam] = _evs
    _EVENTS[(_l1, _l2)] = _per

_STEPS = [(ch, pr) for ch in range(2) for pr in range(len(_PAIR_LIST))]
_CHUNK_WORDS = 2048

def _pair_sections(l1, l2):
    secs = []
    off = 0
    for lam in _lams_of(l1, l2):
        if lam == 0:
            continue
        for k in range(2 * lam + 1):
            secs.append((lam, k, off))
            off += _CHUNK_WORDS
    return secs

_MAX_CHUNK = max(
    len(_pair_sections(l1, l2)) for (l1, l2) in _PAIR_LIST) * _CHUNK_WORDS


def _tec_kernel(in_ref, oe0, oe1, oe2, oe3, oo0, oo1, oo2, oo3,
                in_v, chunk0, chunk1, lam0a, lam0b, dma_sem):
    out_even = [oe0, oe1, oe2, oe3]
    out_odd = [oo0, oo1, oo2, oo3]
    chunks = [chunk0, chunk1]
    lam0_bufs = [lam0a, lam0b]

    cid = lax.axis_index("c")
    sid = lax.axis_index("s")
    wid = sid * 2 + cid
    e_tile = wid // 2
    c_half = wid % 2
    env_base = e_tile * 8

    pltpu.sync_copy(in_ref.at[pl.ds(env_base * 1024, 8 * 1024)], in_v)

    et_off = [e_tile * _FT[lam] * 1024 for lam in range(4)]
    ch_off = [c_half * _N_PAIRS[lam] * 2048 for lam in range(4)]
    a_off = c_half * 256

    def make_pair_body(l1, l2, parity, buf, lam0_v):
        evs_by_lamk = {}
        for lam in _lams_of(l1, l2):
            for (ke, m, n, c) in _EVENTS[(l1, l2)][lam]:
                evs_by_lamk.setdefault((lam, ke), []).append((m, n, c))
        sec_off = {(lam, k): off for (lam, k, off) in _pair_sections(l1, l2)}
        p_lam0 = None
        if 0 in _lams_of(l1, l2):
            p_lam0 = _PAIRS_LAM[0].index((l1, l2))
        if parity == 0:
            b_par = a_off
        else:
            b_par = 256 - a_off

        def f_body(f, e_local):
            in_base = e_local * 1024
            a_row = in_v[pl.ds(in_base + a_off + f * 16, 16)]
            a_sp = {}
            for m in range(2 * l1 + 1):
                a_sp[m] = jnp.full((16,), a_row[_MOFF[l1] + m], jnp.float32)
            b_base = in_base + 512 + b_par
            b_rows = [in_v[pl.ds(b_base + (_MOFF[l2] + n) * 16, 16)]
                      for n in range(2 * l2 + 1)]
            prod = {}
            ft = f // 8
            fo = (f % 8) * 16
            for lam in _lams_of(l1, l2):
                for k in range(2 * lam + 1):
                    acc = None
                    for (m, n, c) in evs_by_lamk[(lam, k)]:
                        t = prod.get((m, n))
                        if t is None:
                            t = a_sp[m] * b_rows[n]
                            prod[(m, n)] = t
                        term = t * np.float32(c)
                        acc = term if acc is None else acc + term
                    if lam == 0:
                        lam0_v[e_local, pl.ds(p_lam0 * 256 + f * 16, 16)] = acc
                    else:
                        off = sec_off[(lam, k)] + (ft * 8 + e_local) * 128 + fo
                        buf[pl.ds(off, 16)] = acc
            return e_local

        def e_body(e_local, carry):
            plsc.parallel_loop(0, 16, carry=e_local)(f_body)
            return carry

        plsc.parallel_loop(0, 8, carry=jnp.int32(0))(e_body)

    def issue_pair_dmas(l1, l2, parity, buf):
        outs = out_even if parity == 0 else out_odd
        for (lam, k, off) in _pair_sections(l1, l2):
            pos = _PAIRS_LAM[lam].index((l1, l2))
            dst = (k * 16 * _FT[lam] + 2 * pos) * 1024 + et_off[lam] + ch_off[lam]
            pltpu.async_copy(
                buf.at[pl.ds(off, _CHUNK_WORDS)],
                outs[lam].at[pl.ds(dst, _CHUNK_WORDS)],
                dma_sem)

    def drain_pair(l1, l2):
        n = len(_pair_sections(l1, l2))
        for _ in range(n):
            pltpu.make_async_copy(
                out_even[3].at[pl.ds(0, _CHUNK_WORDS)],
                chunk0.at[pl.ds(0, _CHUNK_WORDS)], dma_sem).wait()

    for i, (parity, pr) in enumerate(_STEPS):
        l1, l2 = _PAIR_LIST[pr]
        if i >= 2:
            pl1, pl2 = _PAIR_LIST[_STEPS[i - 2][1]]
            drain_pair(pl1, pl2)
        buf = chunks[i % 2]
        make_pair_body(l1, l2, parity, buf, lam0_bufs[parity])
        issue_pair_dmas(l1, l2, parity, buf)
    for i in (len(_STEPS) - 2, len(_STEPS) - 1):
        pl1, pl2 = _PAIR_LIST[_STEPS[i][1]]
        drain_pair(pl1, pl2)

    for e_local in range(8):
        dst = (env_base + e_local) * 2048 + c_half * 1024
        pltpu.async_copy(lam0a.at[e_local], oe0.at[pl.ds(dst, 1024)], dma_sem)
        pltpu.async_copy(lam0b.at[e_local], oo0.at[pl.ds(dst, 1024)], dma_sem)
    for _ in range(16):
        pltpu.make_async_copy(
            oe0.at[pl.ds(0, 1024)], lam0a.at[0], dma_sem).wait()


@jax.jit
def _run_sc(packed):
    mesh = plsc.VectorSubcoreMesh(core_axis_name="c", subcore_axis_name="s")
    out_type = tuple(
        jax.ShapeDtypeStruct(((2 * lam + 1) * N_ENV * _F_SIZES[lam],), jnp.float32)
        for lam in range(4)) * 2
    scratch = [
        pltpu.VMEM((8 * 1024,), jnp.float32),
        pltpu.VMEM((_MAX_CHUNK,), jnp.float32),
        pltpu.VMEM((_MAX_CHUNK,), jnp.float32),
        pltpu.VMEM((8, 1024), jnp.float32),
        pltpu.VMEM((8, 1024), jnp.float32),
        pltpu.SemaphoreType.DMA,
    ]
    fn = pl.kernel(_tec_kernel, out_type=out_type, mesh=mesh,
                   scratch_types=scratch,
                   compiler_params=pltpu.CompilerParams(
                       needs_layout_passes=False))
    return fn(packed.reshape(-1))


def kernel(first_even_0, first_even_1, first_even_2, first_even_3,
           first_odd_0, first_odd_1, first_odd_2, first_odd_3,
           second_even_0, second_even_1, second_even_2, second_even_3,
           second_odd_0, second_odd_1, second_odd_2, second_odd_3):
    first_e = jnp.concatenate(
        [first_even_0, first_even_1, first_even_2, first_even_3], axis=2)
    first_o = jnp.concatenate(
        [first_odd_0, first_odd_1, first_odd_2, first_odd_3], axis=2)
    sec_e = jnp.concatenate(
        [second_even_0, second_even_1, second_even_2, second_even_3],
        axis=2).transpose(0, 2, 1)
    sec_o = jnp.concatenate(
        [second_odd_0, second_odd_1, second_odd_2, second_odd_3],
        axis=2).transpose(0, 2, 1)
    packed = jnp.concatenate(
        [first_e.reshape(N_ENV, 256), first_o.reshape(N_ENV, 256),
         sec_e.reshape(N_ENV, 256), sec_o.reshape(N_ENV, 256)], axis=1)

    outs = _run_sc(packed)
    res = [None] * 8
    for half in range(2):
        res[half * 4] = outs[half * 4].reshape(N_ENV, _F_SIZES[0], 1)
        for lam in range(1, 4):
            F = _F_SIZES[lam]
            K = 2 * lam + 1
            x5 = outs[half * 4 + lam].reshape(K, 16, F // 128, 8, 128)
            res[half * 4 + lam] = x5.transpose(1, 3, 2, 4, 0).reshape(
                N_ENV, F, K)
    return tuple(res)
